# Initial kernel scaffold; baseline (speedup 1.0000x reference)
import jax
import jax.numpy as jnp
from jax.experimental import pallas as pl


def kernel(x, edge_index, edge_type, basis1, comp1, root1, bias1, basis2, comp2, root2, bias2):
    raise NotImplementedError("write your pallas kernel here")


# speedup vs baseline: 1.0556x; 1.0556x over previous
import functools

import jax
import jax.numpy as jnp
from jax.experimental import pallas as pl
from jax.experimental.pallas import tpu as pltpu

N = 10000
E = 160000
R = 964
B = 2
IN = 256
HID = 128
OUT = 64


def _mm_body(x_ref, w_ref, o_ref):
    o_ref[...] = jnp.dot(x_ref[...], w_ref[...], preferred_element_type=jnp.float32)


def _matmul(x, w, block_m=2000):
    m, k = x.shape
    _, n = w.shape
    grid = m // block_m
    return pl.pallas_call(
        _mm_body,
        grid=(grid,),
        in_specs=[
            pl.BlockSpec((block_m, k), lambda i: (i, 0)),
            pl.BlockSpec((k, n), lambda i: (0, 0)),
        ],
        out_specs=pl.BlockSpec((block_m, n), lambda i: (i, 0)),
        out_shape=jax.ShapeDtypeStruct((m, n), jnp.float32),
    )(x, w)


def _sparse_part(table, comp, edge_index, edge_type, norm):
    src = edge_index[0]
    dst = edge_index[1]
    d = table.shape[1] // 2
    rows = table[src]
    w = comp[edge_type] * norm[:, None]
    msg = rows[:, :d] * w[:, :1] + rows[:, d:] * w[:, 1:]
    return jax.ops.segment_sum(msg, dst, num_segments=N)


def kernel(x, edge_index, edge_type, basis1, comp1, root1, bias1, basis2, comp2, root2, bias2):
    dst = edge_index[1]
    pair_key = dst * R + edge_type
    counts = jnp.bincount(pair_key, length=N * R)
    norm = 1.0 / jnp.maximum(counts[pair_key], 1).astype(jnp.float32)

    w1 = jnp.concatenate([basis1[0], basis1[1], root1], axis=1)
    y1 = _matmul(x, w1)
    table1, rt1 = y1[:, : 2 * HID], y1[:, 2 * HID :]
    agg1 = _sparse_part(table1, comp1, edge_index, edge_type, norm)
    h = jax.nn.relu(agg1 + rt1 + bias1)

    w2 = jnp.concatenate([basis2[0], basis2[1], root2], axis=1)
    y2 = _matmul(h, w2)
    table2, rt2 = y2[:, : 2 * OUT], y2[:, 2 * OUT :]
    agg2 = _sparse_part(table2, comp2, edge_index, edge_type, norm)
    return agg2 + rt2 + bias2


# speedup vs baseline: 1.3881x; 1.3149x over previous
import functools

import jax
import jax.numpy as jnp
from jax import lax
from jax.experimental import pallas as pl
from jax.experimental.pallas import tpu as pltpu
from jax.experimental.pallas import tpu_sc as plsc

N = 10000
E = 160000
R = 964
IN = 256
HID = 128
OUT = 64

NC = 2
NS = 16
L = 16

EP = 163840
EPT = EP // (NC * NS)
EPT2 = EP // NS
CE = 64
NCH = EPT // CE
CK = 128
NCH2 = EPT2 // CK

KZ = 75360
CHUNK = NS * KZ
NROUND = 4
KPAD = NC * NROUND * CHUNK

NPAD = 10240
AGG_ROWS = NPAD
ZB = 7536
ZR = 32

_MESH = plsc.VectorSubcoreMesh(
    core_axis_name="c", subcore_axis_name="s", num_cores=NC, num_subcores=NS
)


def _zero_1d(buf, nwords):
    z = jnp.zeros((L,), jnp.float32)

    def body(i, _):
        buf[pl.ds(i * L, L)] = z
        return 0

    lax.fori_loop(0, nwords // L, body, 0)


def _counts_body(dst_hbm, et_hbm, counts_hbm, cbuf, zbuf, pubbuf, keyall,
                 dstc, etc, ibuf, ones):
    core = lax.axis_index("c")
    sid = lax.axis_index("s")

    _zero_1d(zbuf, ZB)
    for j in range(CK // L):
        ones[pl.ds(j * L, L)] = jnp.ones((L,), jnp.float32)

    toff = sid * EPT2

    def keybody(i, _):
        pltpu.sync_copy(dst_hbm.at[pl.ds(toff + i * CK, CK)], dstc)
        pltpu.sync_copy(et_hbm.at[pl.ds(toff + i * CK, CK)], etc)
        for j in range(CK // L):
            d16 = dstc[pl.ds(j * L, L)]
            e16 = etc[pl.ds(j * L, L)]
            keyall[pl.ds(i * CK + j * L, L)] = d16 * R + e16
        return 0

    lax.fori_loop(0, NCH2, keybody, 0)

    def round_body(r, _):
        base = (core * NROUND + r) * CHUNK

        def zloop(i, _):
            pltpu.sync_copy(zbuf, cbuf.at[pl.ds(sid * KZ + i * ZB, ZB)])
            return 0

        lax.fori_loop(0, KZ // ZB, zloop, 0)
        plsc.subcore_barrier()

        def cloop(i, _):
            for j in range(CK // L):
                k16 = keyall[pl.ds(i * CK + j * L, L)]
                rel = k16 - base
                inr = (rel >= 0) & (rel < CHUNK)
                ibuf[pl.ds(j * L, L)] = jnp.where(inr, rel, CHUNK)
            pltpu.sync_copy(ones, cbuf.at[ibuf], add=True)
            return 0

        lax.fori_loop(0, NCH2, cloop, 0)
        pltpu.sync_copy(cbuf.at[pl.ds(sid * KZ, L)], pubbuf.at[pl.ds(0, L)])
        plsc.subcore_barrier()

        def ploop(i, _):
            off = sid * KZ + i * ZB
            pltpu.sync_copy(cbuf.at[pl.ds(off, ZB)], pubbuf)
            pltpu.sync_copy(pubbuf, counts_hbm.at[pl.ds(base + off, ZB)])
            return 0

        lax.fori_loop(0, KZ // ZB, ploop, 0)
        plsc.subcore_barrier()
        return 0

    lax.fori_loop(0, NROUND, round_body, 0)


_counts_call = pl.kernel(
    _counts_body,
    out_type=jax.ShapeDtypeStruct((KPAD,), jnp.float32),
    mesh=_MESH,
    compiler_params=pltpu.CompilerParams(needs_layout_passes=False),
    scratch_types=[
        pltpu.VMEM_SHARED((CHUNK + L,), jnp.float32),
        pltpu.VMEM((ZB,), jnp.float32),
        pltpu.VMEM((ZB,), jnp.float32),
        pltpu.VMEM((EPT2,), jnp.int32),
        pltpu.VMEM((CK,), jnp.int32),
        pltpu.VMEM((CK,), jnp.int32),
        pltpu.VMEM((CK,), jnp.int32),
        pltpu.VMEM((CK,), jnp.float32),
    ],
)


DP = 128


def _msg_body(src_hbm, dst_hbm, et_hbm, comp0_hbm, comp1_hbm, nsrc_hbm,
              table_hbm, agg_hbm, norm_hbm, aggsh, zrow, dstc, etc, normbuf,
              w0all, w1all, keybuf, srcidx, dstidx, cntbuf, rowbuf, msgbuf,
              comp0v, comp1v, gsem, *, d, first_layer):
    core = lax.axis_index("c")
    sid = lax.axis_index("s")
    g = core * NS + sid
    goff = g * EPT

    for i in range(ZR):
        for j in range(DP // L):
            zrow[i, pl.ds(j * L, L)] = jnp.zeros((L,), jnp.float32)

    def mzloop(i, _):
        for j in range(DP // L):
            msgbuf[i, pl.ds(j * L, L)] = jnp.zeros((L,), jnp.float32)
        return 0

    lax.fori_loop(0, CE, mzloop, 0)
    rows_per_tile = AGG_ROWS // NS

    def zloop(i, _):
        pltpu.sync_copy(zrow, aggsh.at[pl.ds(sid * rows_per_tile + i * ZR, ZR)])
        return 0

    lax.fori_loop(0, rows_per_tile // ZR, zloop, 0)

    pltpu.sync_copy(comp0_hbm, comp0v)
    pltpu.sync_copy(comp1_hbm, comp1v)

    def prep(c, _):
        off = goff + c * CE
        pltpu.sync_copy(et_hbm.at[pl.ds(off, CE)], etc)
        if first_layer:
            pltpu.sync_copy(dst_hbm.at[pl.ds(off, CE)], dstc)
            for j in range(CE // L):
                d16 = dstc[pl.ds(j * L, L)]
                e16 = etc[pl.ds(j * L, L)]
                keybuf[pl.ds(j * L, L)] = d16 * R + e16
            pltpu.async_copy(nsrc_hbm.at[keybuf], cntbuf, gsem).wait()
            for j in range(CE // L):
                c16 = cntbuf[pl.ds(j * L, L)]
                normbuf[pl.ds(j * L, L)] = 1.0 / jnp.maximum(c16, 1.0)
            pltpu.sync_copy(normbuf, norm_hbm.at[pl.ds(off, CE)])
        else:
            pltpu.sync_copy(nsrc_hbm.at[pl.ds(off, CE)], normbuf)
        for j in range(CE // L):
            e16 = etc[pl.ds(j * L, L)]
            n16 = normbuf[pl.ds(j * L, L)]
            w0all[pl.ds(c * CE + j * L, L)] = plsc.load_gather(comp0v, [e16]) * n16
            w1all[pl.ds(c * CE + j * L, L)] = plsc.load_gather(comp1v, [e16]) * n16
        return 0

    lax.fori_loop(0, NCH, prep, 0)
    plsc.subcore_barrier()

    def mloop(c, _):
        pltpu.sync_copy(src_hbm.at[pl.ds(goff + c * CE, CE)], srcidx)
        pltpu.sync_copy(dst_hbm.at[pl.ds(goff + c * CE, CE)], dstidx)
        pltpu.async_copy(table_hbm.at[srcidx], rowbuf, gsem).wait()

        def eloop(e, _):
            sel = jnp.full((L,), c * CE + e, jnp.int32)
            w0 = plsc.load_gather(w0all, [sel])
            w1 = plsc.load_gather(w1all, [sel])
            for j in range(d // L):
                a = rowbuf[e, pl.ds(j * L, L)]
                b = rowbuf[e, pl.ds(d + j * L, L)]
                msgbuf[e, pl.ds(j * L, L)] = a * w0 + b * w1
            return 0

        lax.fori_loop(0, CE, eloop, 0)
        pltpu.sync_copy(msgbuf, aggsh.at[dstidx], add=True)
        return 0

    lax.fori_loop(0, NCH, mloop, 0)
    pltpu.sync_copy(aggsh.at[pl.ds(sid * (AGG_ROWS // NS), 1)],
                    msgbuf.at[pl.ds(0, 1)])
    plsc.subcore_barrier()

    prows = AGG_ROWS // NS

    def ploop(i, _):
        off = sid * prows + i * CE
        pltpu.sync_copy(aggsh.at[pl.ds(off, CE)], msgbuf)
        pltpu.sync_copy(msgbuf, agg_hbm.at[core, pl.ds(off, CE)])
        return 0

    lax.fori_loop(0, prows // CE, ploop, 0)


def _make_msg_call(d, first_layer):
    body = functools.partial(_msg_body, d=d, first_layer=first_layer)
    return pl.kernel(
        body,
        out_type=(
            jax.ShapeDtypeStruct((NC, NPAD, DP), jnp.float32),
            jax.ShapeDtypeStruct((EP,), jnp.float32),
        ),
        mesh=_MESH,
        compiler_params=pltpu.CompilerParams(needs_layout_passes=False),
        scratch_types=[
            pltpu.VMEM_SHARED((AGG_ROWS, DP), jnp.float32),
            pltpu.VMEM((ZR, DP), jnp.float32),
            pltpu.VMEM((CE,), jnp.int32),
            pltpu.VMEM((CE,), jnp.int32),
            pltpu.VMEM((CE,), jnp.float32),
            pltpu.VMEM((EPT,), jnp.float32),
            pltpu.VMEM((EPT,), jnp.float32),
            pltpu.VMEM((CE,), jnp.int32),
            pltpu.VMEM((CE,), jnp.int32),
            pltpu.VMEM((CE,), jnp.int32),
            pltpu.VMEM((CE,), jnp.float32),
            pltpu.VMEM((CE, 2 * d), jnp.float32),
            pltpu.VMEM((CE, DP), jnp.float32),
            pltpu.VMEM((R,), jnp.float32),
            pltpu.VMEM((R,), jnp.float32),
            pltpu.SemaphoreType.DMA,
        ],
    )


_msg_l1 = _make_msg_call(HID, True)
_msg_l2 = _make_msg_call(OUT, False)


def _mm_split_body(x_ref, w_ref, t_ref, r_ref, *, split):
    y = jnp.dot(x_ref[...], w_ref[...], preferred_element_type=jnp.float32)
    t_ref[...] = y[:, :split]
    r_ref[...] = y[:, split:]


def _mm_split(x, w, split, block_m=2000):
    m, k = x.shape
    n = w.shape[1]
    return pl.pallas_call(
        functools.partial(_mm_split_body, split=split),
        grid=(m // block_m,),
        in_specs=[
            pl.BlockSpec((block_m, k), lambda i: (i, 0)),
            pl.BlockSpec((k, n), lambda i: (0, 0)),
        ],
        out_specs=[
            pl.BlockSpec((block_m, split), lambda i: (i, 0)),
            pl.BlockSpec((block_m, n - split), lambda i: (i, 0)),
        ],
        out_shape=[
            jax.ShapeDtypeStruct((m, split), jnp.float32),
            jax.ShapeDtypeStruct((m, n - split), jnp.float32),
        ],
    )(x, w)


def _comb_mm_body(a_ref, rt_ref, b_ref, w_ref, t_ref, r_ref, *, split):
    h = jnp.maximum(a_ref[0] + a_ref[1] + rt_ref[...] + b_ref[...], 0.0)
    y = jnp.dot(h, w_ref[...], preferred_element_type=jnp.float32)
    t_ref[...] = y[:, :split]
    r_ref[...] = y[:, split:]


def _comb_mm(aggp, rt, bias2d, w, split, block_m=2000):
    m, k = rt.shape
    n = w.shape[1]
    return pl.pallas_call(
        functools.partial(_comb_mm_body, split=split),
        grid=(m // block_m,),
        in_specs=[
            pl.BlockSpec((NC, block_m, k), lambda i: (0, i, 0)),
            pl.BlockSpec((block_m, k), lambda i: (i, 0)),
            pl.BlockSpec((1, k), lambda i: (0, 0)),
            pl.BlockSpec((k, n), lambda i: (0, 0)),
        ],
        out_specs=[
            pl.BlockSpec((block_m, split), lambda i: (i, 0)),
            pl.BlockSpec((block_m, n - split), lambda i: (i, 0)),
        ],
        out_shape=[
            jax.ShapeDtypeStruct((m, split), jnp.float32),
            jax.ShapeDtypeStruct((m, n - split), jnp.float32),
        ],
    )(aggp, rt, bias2d, w)


def _final_body(a_ref, rt_ref, b_ref, o_ref):
    k = rt_ref.shape[1]
    o_ref[...] = a_ref[0, :, :k] + a_ref[1, :, :k] + rt_ref[...] + b_ref[...]


def _final(aggp, rt, bias2d, block_m=2000):
    m, k = rt.shape
    return pl.pallas_call(
        _final_body,
        grid=(m // block_m,),
        in_specs=[
            pl.BlockSpec((NC, block_m, aggp.shape[2]), lambda i: (0, i, 0)),
            pl.BlockSpec((block_m, k), lambda i: (i, 0)),
            pl.BlockSpec((1, k), lambda i: (0, 0)),
        ],
        out_specs=pl.BlockSpec((block_m, k), lambda i: (i, 0)),
        out_shape=jax.ShapeDtypeStruct((m, k), jnp.float32),
    )(aggp, rt, bias2d)


def kernel(x, edge_index, edge_type, basis1, comp1, root1, bias1, basis2,
           comp2, root2, bias2):
    pad = EP - E
    src_p = jnp.concatenate([edge_index[0], jnp.zeros((pad,), jnp.int32)])
    dst_p = jnp.concatenate([edge_index[1], jnp.full((pad,), N, jnp.int32)])
    et_p = jnp.concatenate([edge_type, jnp.zeros((pad,), jnp.int32)])

    counts = _counts_call(dst_p, et_p)

    w1 = jnp.concatenate([basis1[0], basis1[1], root1], axis=1)
    table1, rt1 = _mm_split(x, w1, 2 * HID)
    agg1p, norm = _msg_l1(src_p, dst_p, et_p,
                          comp1[:, 0] + 0.0,
                          comp1[:, 1] + 0.0, counts, table1)

    w2 = jnp.concatenate([basis2[0], basis2[1], root2], axis=1)
    table2, rt2 = _comb_mm(agg1p, rt1, bias1.reshape(1, HID), w2, 2 * OUT)

    agg2p, _ = _msg_l2(src_p, dst_p, et_p,
                       comp2[:, 0] + 0.0,
                       comp2[:, 1] + 0.0, norm, table2)

    return _final(agg2p, rt2, bias2.reshape(1, OUT))
